# baseline (speedup 1.0000x reference)
import functools

import jax
import jax.numpy as jnp
from jax import lax
from jax.experimental import pallas as pl
from jax.experimental.pallas import tpu as pltpu
from jax.experimental.pallas import tpu_sc as plsc

NC = 2
NS = 16
L = 16
NW = NC * NS

BATCH_ = 16384
EMB_ = 64
BPW = BATCH_ // NW
CHUNK = 128
NCHUNK = BPW // CHUNK


def _body(uid_hbm, mid_hbm, ut_hbm, mt_hbm, out_hbm,
          uidx, midx, urows, mrows, outv, sems):
    wid = lax.axis_index("s") * NC + lax.axis_index("c")

    pltpu.sync_copy(uid_hbm.at[wid], uidx)
    pltpu.sync_copy(mid_hbm.at[wid], midx)

    copies = []
    for j in range(NCHUNK):
        rows_dst = urows.at[pl.ds(j * CHUNK, CHUNK), :]
        copies.append(pltpu.async_copy(ut_hbm.at[uidx.at[j]], rows_dst,
                                       sems[2 * j]))
        rows_dst = mrows.at[pl.ds(j * CHUNK, CHUNK), :]
        copies.append(pltpu.async_copy(mt_hbm.at[midx.at[j]], rows_dst,
                                       sems[2 * j + 1]))

    lane = lax.iota(jnp.int32, L)

    def group_body(g, _):
        rows = g * L + lane
        def d_body(d, acc):
            col = jnp.full((L,), d, jnp.int32)
            u = plsc.load_gather(urows, [rows, col])
            m = plsc.load_gather(mrows, [rows, col])
            return acc + u * m
        acc = lax.fori_loop(0, EMB_, d_body, jnp.zeros((L,), jnp.float32))
        outv[pl.ds(g * L, L)] = acc
        return 0

    for j in range(NCHUNK):
        copies[2 * j].wait()
        copies[2 * j + 1].wait()
        lax.fori_loop(j * (CHUNK // L), (j + 1) * (CHUNK // L), group_body, 0)

    pltpu.sync_copy(outv, out_hbm.at[wid])


@jax.jit
def _mf_dot(user_id, movie_id, user_table, movie_table):
    mesh = plsc.VectorSubcoreMesh(core_axis_name="c", subcore_axis_name="s")
    uid = user_id.astype(jnp.int32).reshape(NW, NCHUNK, CHUNK)
    mid = movie_id.astype(jnp.int32).reshape(NW, NCHUNK, CHUNK)
    out = pl.kernel(
        _body,
        out_type=jax.ShapeDtypeStruct((NW, BPW), jnp.float32),
        mesh=mesh,
        compiler_params=pltpu.CompilerParams(
            needs_layout_passes=False, use_tc_tiling_on_sc=False),
        scratch_types=[
            pltpu.VMEM((NCHUNK, CHUNK), jnp.int32),
            pltpu.VMEM((NCHUNK, CHUNK), jnp.int32),
            pltpu.VMEM((BPW, EMB_), jnp.float32),
            pltpu.VMEM((BPW, EMB_), jnp.float32),
            pltpu.VMEM((BPW,), jnp.float32),
            [pltpu.SemaphoreType.DMA] * (2 * NCHUNK),
        ],
    )(uid, mid, user_table, movie_table)
    return out.reshape(BATCH_)


def kernel(user_id, movie_id, user_table, movie_table):
    return _mf_dot(user_id, movie_id, user_table, movie_table)
